# baseline (speedup 1.0000x reference)
import functools

import jax
import jax.numpy as jnp
from jax import lax
from jax.experimental import pallas as pl
from jax.experimental.pallas import tpu as pltpu
from jax.experimental.pallas import tpu_sc as plsc

_NUM_CLASS = 1000
_C, _H, _W = 3, 256, 128
_BATCH = 256
_NC, _NS = 2, 16
_NW = _NC * _NS
_ROWS_PER_W = _BATCH // _NW
_HB = 128
_BPC = _H // _HB
_G = _ROWS_PER_W * _C * _BPC
_NB = 6
_LANES = 16

_mesh = plsc.VectorSubcoreMesh(core_axis_name="c", subcore_axis_name="s")


@functools.partial(
    pl.kernel,
    mesh=_mesh,
    out_type=jax.ShapeDtypeStruct((_BATCH, _C, _H, _W), jnp.float32),
    scratch_types=(
        [pltpu.VMEM((_LANES,), jnp.int32)]
        + [pltpu.VMEM((_HB, _W), jnp.float32) for _ in range(_NB)]
        + [pltpu.SemaphoreType.DMA for _ in range(2 * _NB)]
    ),
)
def _gather_rows(tbl_hbm, lab_hbm, out_hbm, lab_v, *bufs_and_sems):
    bufs = bufs_and_sems[:_NB]
    sgs = bufs_and_sems[_NB:2 * _NB]
    sws = bufs_and_sems[2 * _NB:]

    wid = lax.axis_index("s") * _NC + lax.axis_index("c")
    lab_base = wid * _ROWS_PER_W

    pltpu.sync_copy(lab_hbm.at[pl.ds(lab_base, _ROWS_PER_W)],
                    lab_v.at[pl.ds(0, _ROWS_PER_W)])
    labs = lab_v[...]
    lab_s = [labs[t] for t in range(_ROWS_PER_W)]

    def start_gather(g):
        t, c, h = g // (_C * _BPC), (g // _BPC) % _C, (g % _BPC) * _HB
        return pltpu.async_copy(
            tbl_hbm.at[lab_s[t], c, pl.ds(h, _HB)],
            bufs[g % _NB], sgs[g % _NB])

    def start_write(g):
        t, c, h = g // (_C * _BPC), (g // _BPC) % _C, (g % _BPC) * _HB
        return pltpu.async_copy(
            bufs[g % _NB],
            out_hbm.at[lab_base + t, c, pl.ds(h, _HB)],
            sws[g % _NB])

    hw = [None] * _G
    hg = [None] * _G
    for g in range(_NB - 1):
        hg[g] = start_gather(g)
    for g in range(_G):
        hg[g].wait()
        if g + _NB - 1 < _G:
            if g >= 1:
                hw[g - 1].wait()
            hg[g + _NB - 1] = start_gather(g + _NB - 1)
        hw[g] = start_write(g)
    for g in range(_G - _NB, _G):
        hw[g].wait()


def kernel(label, learnable_person_info):
    return _gather_rows(learnable_person_info, label)


# speedup vs baseline: 1.0076x; 1.0076x over previous
import functools

import jax
import jax.numpy as jnp
from jax import lax
from jax.experimental import pallas as pl
from jax.experimental.pallas import tpu as pltpu
from jax.experimental.pallas import tpu_sc as plsc

_NUM_CLASS = 1000
_C, _H, _W = 3, 256, 128
_BATCH = 256
_R = _C * _H
_NC, _NS = 2, 16
_NW = _NC * _NS
_ROWS_PER_W = _BATCH // _NW
_HB = 192
_BPL = _R // _HB
_G = _ROWS_PER_W * _BPL
_NB = 4
_LANES = 16

_mesh = plsc.VectorSubcoreMesh(core_axis_name="c", subcore_axis_name="s")


@functools.partial(
    pl.kernel,
    mesh=_mesh,
    out_type=jax.ShapeDtypeStruct((_BATCH, _R, _W), jnp.float32),
    scratch_types=(
        [pltpu.VMEM((_LANES,), jnp.int32)]
        + [pltpu.VMEM((_HB, _W), jnp.float32) for _ in range(_NB)]
        + [pltpu.SemaphoreType.DMA for _ in range(2 * _NB)]
    ),
)
def _gather_rows(tbl_hbm, lab_hbm, out_hbm, lab_v, *bufs_and_sems):
    bufs = bufs_and_sems[:_NB]
    sgs = bufs_and_sems[_NB:2 * _NB]
    sws = bufs_and_sems[2 * _NB:]

    wid = lax.axis_index("s") * _NC + lax.axis_index("c")
    lab_base = wid * _ROWS_PER_W

    pltpu.sync_copy(lab_hbm.at[pl.ds(lab_base, _ROWS_PER_W)],
                    lab_v.at[pl.ds(0, _ROWS_PER_W)])
    labs = lab_v[...]
    lab_s = [labs[t] for t in range(_ROWS_PER_W)]

    def start_gather(g):
        t, h = g // _BPL, (g % _BPL) * _HB
        return pltpu.async_copy(
            tbl_hbm.at[lab_s[t], pl.ds(h, _HB)],
            bufs[g % _NB], sgs[g % _NB])

    def start_write(g):
        t, h = g // _BPL, (g % _BPL) * _HB
        return pltpu.async_copy(
            bufs[g % _NB],
            out_hbm.at[lab_base + t, pl.ds(h, _HB)],
            sws[g % _NB])

    hw = [None] * _G
    hg = [None] * _G
    for g in range(_NB - 1):
        hg[g] = start_gather(g)
    for g in range(_G):
        hg[g].wait()
        if g + _NB - 1 < _G:
            if g >= 1:
                hw[g - 1].wait()
            hg[g + _NB - 1] = start_gather(g + _NB - 1)
        hw[g] = start_write(g)
    for g in range(_G - _NB, _G):
        hw[g].wait()


def kernel(label, learnable_person_info):
    tbl = learnable_person_info.reshape(_NUM_CLASS, _R, _W)
    out = _gather_rows(tbl, label)
    return out.reshape(_BATCH, _C, _H, _W)


# speedup vs baseline: 1.0129x; 1.0052x over previous
import functools

import jax
import jax.numpy as jnp
from jax import lax
from jax.experimental import pallas as pl
from jax.experimental.pallas import tpu as pltpu
from jax.experimental.pallas import tpu_sc as plsc

_NUM_CLASS = 1000
_C, _H, _W = 3, 256, 128
_BATCH = 256
_R = _C * _H
_NC, _NS = 2, 16
_NW = _NC * _NS
_ROWS_PER_W = _BATCH // _NW
_HB = 384
_BPL = _R // _HB
_G = _ROWS_PER_W * _BPL
_NB = 2
_LANES = 16

_mesh = plsc.VectorSubcoreMesh(core_axis_name="c", subcore_axis_name="s")


@functools.partial(
    pl.kernel,
    mesh=_mesh,
    out_type=jax.ShapeDtypeStruct((_BATCH, _R, _W), jnp.float32),
    scratch_types=(
        [pltpu.VMEM((_LANES,), jnp.int32)]
        + [pltpu.VMEM((_HB, _W), jnp.float32) for _ in range(_NB)]
        + [pltpu.SemaphoreType.DMA for _ in range(2 * _NB)]
    ),
)
def _gather_rows(tbl_hbm, lab_hbm, out_hbm, lab_v, *bufs_and_sems):
    bufs = bufs_and_sems[:_NB]
    sgs = bufs_and_sems[_NB:2 * _NB]
    sws = bufs_and_sems[2 * _NB:]

    wid = lax.axis_index("s") * _NC + lax.axis_index("c")
    lab_base = wid * _ROWS_PER_W

    pltpu.sync_copy(lab_hbm.at[pl.ds(lab_base, _ROWS_PER_W)],
                    lab_v.at[pl.ds(0, _ROWS_PER_W)])
    labs = lab_v[...]
    lab_s = [labs[t] for t in range(_ROWS_PER_W)]

    def start_gather(g):
        t, h = g // _BPL, (g % _BPL) * _HB
        return pltpu.async_copy(
            tbl_hbm.at[lab_s[t], pl.ds(h, _HB)],
            bufs[g % _NB], sgs[g % _NB])

    def start_write(g):
        t, h = g // _BPL, (g % _BPL) * _HB
        return pltpu.async_copy(
            bufs[g % _NB],
            out_hbm.at[lab_base + t, pl.ds(h, _HB)],
            sws[g % _NB])

    hw = [None] * _G
    hg = [None] * _G
    for g in range(_NB - 1):
        hg[g] = start_gather(g)
    for g in range(_G):
        hg[g].wait()
        if g + _NB - 1 < _G:
            if g >= 1:
                hw[g - 1].wait()
            hg[g + _NB - 1] = start_gather(g + _NB - 1)
        hw[g] = start_write(g)
    for g in range(_G - _NB, _G):
        hw[g].wait()


def kernel(label, learnable_person_info):
    tbl = learnable_person_info.reshape(_NUM_CLASS, _R, _W)
    out = _gather_rows(tbl, label)
    return out.reshape(_BATCH, _C, _H, _W)
